# Initial kernel scaffold; baseline (speedup 1.0000x reference)
import jax
import jax.numpy as jnp
from jax.experimental import pallas as pl


def kernel(features, vertices, edges, faces, W, b):
    raise NotImplementedError("write your pallas kernel here")


# speedup vs baseline: 11.3398x; 11.3398x over previous
import jax
import jax.numpy as jnp
from jax import lax
from jax.experimental import pallas as pl
from jax.experimental.pallas import tpu as pltpu
from jax.experimental.pallas import tpu_sc as plsc

N = 10000
E = 320000
C = 128
NPAD = 10240
NC, NS, L = 2, 16, 16
NW = NC * NS
EC = E // NW
K = 79
EPT = K * 128
RPT = NPAD // NS


def _rsqrt(x):
    i = plsc.bitcast(x, jnp.int32)
    i = jnp.int32(0x5F3759DF) - lax.shift_right_logical(i, 1)
    y = plsc.bitcast(i, jnp.float32)
    for _ in range(3):
        y = y * (1.5 - 0.5 * x * y * y)
    return y


def _sc_body(vx_h, vy_h, vz_h, src_h, dst_h, xw_h, out_h, w_hbm,
             dgo_sh, dgi_sh, out_sh, sem):
    c = lax.axis_index("c")
    s = lax.axis_index("s")
    zv = jnp.zeros((L,), jnp.float32)

    if True:
        def phase0(zbd, zb2):
            @pl.loop(0, 640 // L)
            def _(i):
                zbd[pl.ds(i * L, L)] = zv

            @pl.loop(0, 40)
            def _(r):
                for k in range(C // L):
                    zb2[r, pl.ds(k * L, L)] = zv

            pltpu.sync_copy(zbd, dgo_sh.at[pl.ds(640 * s, 640)])
            pltpu.sync_copy(zbd, dgi_sh.at[pl.ds(640 * s, 640)])
            for i in range(RPT // 40):
                pltpu.sync_copy(zb2, out_sh.at[pl.ds(RPT * s + 40 * i, 40)])

        pl.run_scoped(phase0,
                      pltpu.VMEM((640,), jnp.float32),
                      pltpu.VMEM((40, C), jnp.float32))
        plsc.subcore_barrier()

        def phase1(vx_v, vy_v, vz_v, sr, dr, wr):
            pltpu.sync_copy(vx_h, vx_v)
            pltpu.sync_copy(vy_h, vy_v)
            pltpu.sync_copy(vz_h, vz_v)

            def edge_weights(chunk, keep_w):
                @pl.loop(0, K)
                def _(j):
                    pltpu.sync_copy(src_h.at[chunk, j], sr)
                    pltpu.sync_copy(dst_h.at[chunk, j], dr)
                    for k in range(C // L):
                        sl = pl.ds(k * L, L)
                        si = sr[sl]
                        di = dr[sl]
                        dx = (plsc.load_gather(vx_v, [si])
                              - plsc.load_gather(vx_v, [di]))
                        dy = (plsc.load_gather(vy_v, [si])
                              - plsc.load_gather(vy_v, [di]))
                        dz = (plsc.load_gather(vz_v, [si])
                              - plsc.load_gather(vz_v, [di]))
                        ss = dx * dx + dy * dy + dz * dz + 1e-12
                        dist = ss * _rsqrt(ss)
                        w = jnp.exp(-dist)
                        eidx = j * 128 + k * L + lax.iota(jnp.int32, L)
                        w = jnp.where(eidx < EC, w, 0.0)
                        wr[sl] = w
                    pltpu.sync_copy(wr, dgo_sh.at[sr], add=True)
                    pltpu.sync_copy(wr, dgi_sh.at[dr], add=True)
                    if keep_w:
                        pltpu.sync_copy(wr, w_hbm.at[chunk, j])

            edge_weights((1 - c) * NS + s, False)
            edge_weights(c * NS + s, True)

        pl.run_scoped(phase1,
                      pltpu.VMEM((N,), jnp.float32),
                      pltpu.VMEM((N,), jnp.float32),
                      pltpu.VMEM((N,), jnp.float32),
                      pltpu.VMEM((128,), jnp.int32),
                      pltpu.VMEM((128,), jnp.int32),
                      pltpu.VMEM((128,), jnp.float32))
        plsc.subcore_barrier()

        def phase2(dgo_v, dgi_v, rows_v, sr, dr, wr):
            own = c * NS + s
            pltpu.sync_copy(dgo_sh, dgo_v)
            pltpu.sync_copy(dgi_sh, dgi_v)

            @pl.loop(0, K)
            def _(b):
                pltpu.sync_copy(src_h.at[own, b], sr)
                pltpu.sync_copy(dst_h.at[own, b], dr)
                pltpu.sync_copy(w_hbm.at[own, b], wr)
                gather = pltpu.async_copy(xw_h.at[dr], rows_v, sem)
                wn_list = []
                for g in range(128 // L):
                    sl = pl.ds(g * L, L)
                    si = sr[sl]
                    di = dr[sl]
                    p = (plsc.load_gather(dgo_v, [si])
                         * plsc.load_gather(dgi_v, [di]))
                    sq = p * _rsqrt(p)
                    wn_list.append(wr[sl] / (sq + 1e-8))
                gather.wait()
                for g in range(128 // L):
                    wn16 = wn_list[g]
                    for t in range(L):
                        wn = wn16[t]
                        row = g * L + t
                        for k in range(C // L):
                            sl = pl.ds(k * L, L)
                            rows_v[row, sl] = rows_v[row, sl] * wn
                pltpu.sync_copy(rows_v, out_sh.at[sr], add=True)

        pl.run_scoped(phase2,
                      pltpu.VMEM((NPAD,), jnp.float32),
                      pltpu.VMEM((NPAD,), jnp.float32),
                      pltpu.VMEM((128, C), jnp.float32),
                      pltpu.VMEM((128,), jnp.int32),
                      pltpu.VMEM((128,), jnp.int32),
                      pltpu.VMEM((128,), jnp.float32))
        plsc.subcore_barrier()

        pltpu.sync_copy(out_sh.at[pl.ds(RPT * s, RPT)],
                        out_h.at[c].at[pl.ds(RPT * s, RPT)])


def _mm_body(x_ref, w_ref, o_ref):
    o_ref[...] = jnp.dot(x_ref[...], w_ref[...],
                         preferred_element_type=jnp.float32)


def _comb_body(p_ref, b_ref, o_ref):
    o_ref[...] = p_ref[0] + p_ref[1] + b_ref[...]


@jax.jit
def kernel(features, vertices, edges, faces, W, b):
    del faces
    xw = pl.pallas_call(
        _mm_body,
        grid=(10,),
        in_specs=[pl.BlockSpec((N // 10, C), lambda i: (i, 0)),
                  pl.BlockSpec((C, C), lambda i: (0, 0))],
        out_specs=pl.BlockSpec((N // 10, C), lambda i: (i, 0)),
        out_shape=jax.ShapeDtypeStruct((N, C), jnp.float32),
    )(features, W)

    vx = vertices[:, 0]
    vy = vertices[:, 1]
    vz = vertices[:, 2]
    src = jnp.pad(edges[0].reshape(NW, EC), ((0, 0), (0, EPT - EC))
                  ).reshape(NW, K, 128).astype(jnp.int32)
    dst = jnp.pad(edges[1].reshape(NW, EC), ((0, 0), (0, EPT - EC))
                  ).reshape(NW, K, 128).astype(jnp.int32)

    mesh = plsc.VectorSubcoreMesh(core_axis_name="c", subcore_axis_name="s")
    partial, _ = pl.kernel(
        _sc_body,
        out_type=(jax.ShapeDtypeStruct((NC, NPAD, C), jnp.float32),
                  jax.ShapeDtypeStruct((NW, K, 128), jnp.float32)),
        mesh=mesh,
        compiler_params=pltpu.CompilerParams(needs_layout_passes=False),
        scratch_types=[pltpu.VMEM_SHARED((NPAD,), jnp.float32),
                       pltpu.VMEM_SHARED((NPAD,), jnp.float32),
                       pltpu.VMEM_SHARED((NPAD, C), jnp.float32),
                       pltpu.SemaphoreType.DMA],
    )(vx, vy, vz, src, dst, xw)

    out = pl.pallas_call(
        _comb_body,
        grid=(10,),
        in_specs=[pl.BlockSpec((NC, N // 10, C), lambda i: (0, i, 0)),
                  pl.BlockSpec((1, C), lambda i: (0, 0))],
        out_specs=pl.BlockSpec((N // 10, C), lambda i: (i, 0)),
        out_shape=jax.ShapeDtypeStruct((N, C), jnp.float32),
    )(partial, b.reshape(1, C))
    return out


# speedup vs baseline: 11.6761x; 1.0297x over previous
import jax
import jax.numpy as jnp
from jax import lax
from jax.experimental import pallas as pl
from jax.experimental.pallas import tpu as pltpu
from jax.experimental.pallas import tpu_sc as plsc

N = 10000
E = 320000
C = 128
NPAD = 10240
NC, NS, L = 2, 16, 16
NW = NC * NS
EC = E // NW
K = 80
EPT = K * 128
BR = 16
NB = K // BR
RPT = NPAD // NS


def _rsqrt(x):
    i = plsc.bitcast(x, jnp.int32)
    i = jnp.int32(0x5F3759DF) - lax.shift_right_logical(i, 1)
    y = plsc.bitcast(i, jnp.float32)
    for _ in range(3):
        y = y * (1.5 - 0.5 * x * y * y)
    return y


def _sc_body(vx_h, vy_h, vz_h, src_h, dst_h, xw_h, out_h, w_hbm,
             dgo_sh, dgi_sh, out_sh, gsem, ssem0, ssem1):
    c = lax.axis_index("c")
    s = lax.axis_index("s")
    zv = jnp.zeros((L,), jnp.float32)
    own = c * NS + s

    def phase0(zbd, zb2):
        @pl.loop(0, 640 // L)
        def _(i):
            zbd[pl.ds(i * L, L)] = zv

        @pl.loop(0, 40)
        def _(r):
            for k in range(C // L):
                zb2[r, pl.ds(k * L, L)] = zv

        pltpu.sync_copy(zbd, dgo_sh.at[pl.ds(640 * s, 640)])
        pltpu.sync_copy(zbd, dgi_sh.at[pl.ds(640 * s, 640)])
        for i in range(RPT // 40):
            pltpu.sync_copy(zb2, out_sh.at[pl.ds(RPT * s + 40 * i, 40)])

    pl.run_scoped(phase0,
                  pltpu.VMEM((640,), jnp.float32),
                  pltpu.VMEM((40, C), jnp.float32))
    plsc.subcore_barrier()

    def phase1(vx_v, vy_v, vz_v, s16, d16, w16):
        pltpu.sync_copy(vx_h, vx_v)
        pltpu.sync_copy(vy_h, vy_v)
        pltpu.sync_copy(vz_h, vz_v)

        def edge_weights(chunk, keep_w):
            @pl.loop(0, NB)
            def _(b2):
                pltpu.sync_copy(src_h.at[chunk, pl.ds(b2 * BR, BR)], s16)
                pltpu.sync_copy(dst_h.at[chunk, pl.ds(b2 * BR, BR)], d16)

                @pl.loop(0, BR)
                def _(r):
                    for k in range(C // L):
                        sl = pl.ds(k * L, L)
                        si = s16[r, sl]
                        di = d16[r, sl]
                        dx = (plsc.load_gather(vx_v, [si])
                              - plsc.load_gather(vx_v, [di]))
                        dy = (plsc.load_gather(vy_v, [si])
                              - plsc.load_gather(vy_v, [di]))
                        dz = (plsc.load_gather(vz_v, [si])
                              - plsc.load_gather(vz_v, [di]))
                        ss = dx * dx + dy * dy + dz * dz + 1e-12
                        dist = ss * _rsqrt(ss)
                        w = jnp.exp(-dist)
                        eidx = (b2 * BR + r) * 128 + k * L + lax.iota(jnp.int32, L)
                        w = jnp.where(eidx < EC, w, 0.0)
                        w16[r, sl] = w
                    pltpu.sync_copy(w16.at[r], dgo_sh.at[s16.at[r]], add=True)
                    pltpu.sync_copy(w16.at[r], dgi_sh.at[d16.at[r]], add=True)

                if keep_w:
                    pltpu.sync_copy(w16, w_hbm.at[chunk, pl.ds(b2 * BR, BR)])

        edge_weights((1 - c) * NS + s, False)
        edge_weights(own, True)

    pl.run_scoped(phase1,
                  pltpu.VMEM((N,), jnp.float32),
                  pltpu.VMEM((N,), jnp.float32),
                  pltpu.VMEM((N,), jnp.float32),
                  pltpu.VMEM((BR, 128), jnp.int32),
                  pltpu.VMEM((BR, 128), jnp.int32),
                  pltpu.VMEM((BR, 128), jnp.float32))
    plsc.subcore_barrier()

    def phase15(dgo_v, dgi_v, s16, d16, w16):
        pltpu.sync_copy(dgo_sh, dgo_v)
        pltpu.sync_copy(dgi_sh, dgi_v)

        @pl.loop(0, NB)
        def _(b2):
            pltpu.sync_copy(src_h.at[own, pl.ds(b2 * BR, BR)], s16)
            pltpu.sync_copy(dst_h.at[own, pl.ds(b2 * BR, BR)], d16)
            pltpu.sync_copy(w_hbm.at[own, pl.ds(b2 * BR, BR)], w16)

            @pl.loop(0, BR)
            def _(r):
                for k in range(C // L):
                    sl = pl.ds(k * L, L)
                    p = (plsc.load_gather(dgo_v, [s16[r, sl]])
                         * plsc.load_gather(dgi_v, [d16[r, sl]]))
                    sq = p * _rsqrt(p)
                    w16[r, sl] = w16[r, sl] / (sq + 1e-8)

            pltpu.sync_copy(w16, w_hbm.at[own, pl.ds(b2 * BR, BR)])

    pl.run_scoped(phase15,
                  pltpu.VMEM((NPAD,), jnp.float32),
                  pltpu.VMEM((NPAD,), jnp.float32),
                  pltpu.VMEM((BR, 128), jnp.int32),
                  pltpu.VMEM((BR, 128), jnp.int32),
                  pltpu.VMEM((BR, 128), jnp.float32))

    def phase2(rows0, rows1, di0, di1, si0, si1, sc0, sc1, wn0, wn1):
        rows = (rows0, rows1)
        di = (di0, di1)
        si = (si0, si1)
        sci = (sc0, sc1)
        wnb = (wn0, wn1)
        ssem = (ssem0, ssem1)

        pltpu.sync_copy(dst_h.at[own, 0], di0)
        pltpu.sync_copy(src_h.at[own, 0], si0)
        pltpu.sync_copy(w_hbm.at[own, 0], wn0)
        pltpu.async_copy(xw_h.at[di0], rows0, gsem)

        @pl.loop(0, K // 2)
        def _(t):
            for par in range(2):
                b = 2 * t + par

                pltpu.make_async_copy(
                    xw_h.at[di[par]], rows[par], gsem).wait()

                @pl.when(b >= 1)
                def _():
                    pltpu.make_async_copy(
                        rows[1 - par], out_sh.at[sci[1 - par]],
                        ssem[1 - par]).wait()

                @pl.when(b + 1 < K)
                def _():
                    pltpu.sync_copy(dst_h.at[own, b + 1], di[1 - par])
                    pltpu.sync_copy(src_h.at[own, b + 1], si[1 - par])
                    pltpu.sync_copy(w_hbm.at[own, b + 1], wnb[1 - par])
                    pltpu.async_copy(xw_h.at[di[1 - par]], rows[1 - par], gsem)

                for g in range(128 // L):
                    wn16 = wnb[par][pl.ds(g * L, L)]
                    for u in range(L):
                        wn = wn16[u]
                        row = g * L + u
                        for k in range(C // L):
                            sl = pl.ds(k * L, L)
                            rows[par][row, sl] = rows[par][row, sl] * wn

                for k in range(128 // L):
                    sl = pl.ds(k * L, L)
                    sci[par][sl] = si[par][sl]
                pltpu.async_copy(rows[par], out_sh.at[sci[par]],
                                 ssem[par], add=True)

        lastp = (K - 1) % 2
        pltpu.make_async_copy(rows[lastp], out_sh.at[sci[lastp]],
                              ssem[lastp]).wait()

    pl.run_scoped(phase2,
                  pltpu.VMEM((128, C), jnp.float32),
                  pltpu.VMEM((128, C), jnp.float32),
                  pltpu.VMEM((128,), jnp.int32),
                  pltpu.VMEM((128,), jnp.int32),
                  pltpu.VMEM((128,), jnp.int32),
                  pltpu.VMEM((128,), jnp.int32),
                  pltpu.VMEM((128,), jnp.int32),
                  pltpu.VMEM((128,), jnp.int32),
                  pltpu.VMEM((128,), jnp.float32),
                  pltpu.VMEM((128,), jnp.float32))
    plsc.subcore_barrier()

    pltpu.sync_copy(out_sh.at[pl.ds(RPT * s, RPT)],
                    out_h.at[c].at[pl.ds(RPT * s, RPT)])


def _mm_body(x_ref, w_ref, o_ref):
    o_ref[...] = jnp.dot(x_ref[...], w_ref[...],
                         preferred_element_type=jnp.float32)


def _comb_body(p_ref, b_ref, o_ref):
    o_ref[...] = p_ref[0] + p_ref[1] + b_ref[...]


@jax.jit
def kernel(features, vertices, edges, faces, W, b):
    del faces
    xw = pl.pallas_call(
        _mm_body,
        grid=(10,),
        in_specs=[pl.BlockSpec((N // 10, C), lambda i: (i, 0)),
                  pl.BlockSpec((C, C), lambda i: (0, 0))],
        out_specs=pl.BlockSpec((N // 10, C), lambda i: (i, 0)),
        out_shape=jax.ShapeDtypeStruct((N, C), jnp.float32),
    )(features, W)

    vx = vertices[:, 0]
    vy = vertices[:, 1]
    vz = vertices[:, 2]
    src = jnp.pad(edges[0].reshape(NW, EC), ((0, 0), (0, EPT - EC))
                  ).reshape(NW, K, 128).astype(jnp.int32)
    dst = jnp.pad(edges[1].reshape(NW, EC), ((0, 0), (0, EPT - EC))
                  ).reshape(NW, K, 128).astype(jnp.int32)

    mesh = plsc.VectorSubcoreMesh(core_axis_name="c", subcore_axis_name="s")
    partial, _ = pl.kernel(
        _sc_body,
        out_type=(jax.ShapeDtypeStruct((NC, NPAD, C), jnp.float32),
                  jax.ShapeDtypeStruct((NW, K, 128), jnp.float32)),
        mesh=mesh,
        compiler_params=pltpu.CompilerParams(needs_layout_passes=False),
        scratch_types=[pltpu.VMEM_SHARED((NPAD,), jnp.float32),
                       pltpu.VMEM_SHARED((NPAD,), jnp.float32),
                       pltpu.VMEM_SHARED((NPAD, C), jnp.float32),
                       pltpu.SemaphoreType.DMA,
                       pltpu.SemaphoreType.DMA,
                       pltpu.SemaphoreType.DMA],
    )(vx, vy, vz, src, dst, xw)

    out = pl.pallas_call(
        _comb_body,
        grid=(10,),
        in_specs=[pl.BlockSpec((NC, N // 10, C), lambda i: (0, i, 0)),
                  pl.BlockSpec((1, C), lambda i: (0, 0))],
        out_specs=pl.BlockSpec((N // 10, C), lambda i: (i, 0)),
        out_shape=jax.ShapeDtypeStruct((N, C), jnp.float32),
    )(partial, b.reshape(1, C))
    return out
